# baseline (speedup 1.0000x reference)
import functools

import jax
import jax.numpy as jnp
from jax import lax
from jax.experimental import pallas as pl
from jax.experimental.pallas import tpu as pltpu
from jax.experimental.pallas import tpu_sc as plsc

N_NODES = 10000
N_EDGES = 160000
D = 256

NC = 2
NS = 16
NW = NC * NS

@functools.lru_cache(maxsize=None)
def _sc_mesh():
    return plsc.VectorSubcoreMesh(core_axis_name="c", subcore_axis_name="s",
                                  num_cores=NC, num_subcores=NS)


_G_EPW = N_EDGES // NW
_G_K = 128
_G_NCHUNK = _G_EPW // _G_K
_G_TAIL = _G_EPW - _G_NCHUNK * _G_K


def _sc_gather_body(nf_hbm, snd_hbm, rcv_hbm, out_s_hbm, out_r_hbm,
                    idx_v, rows_v, idxt_v, rowst_v, sem):
    wid = lax.axis_index("s") * NC + lax.axis_index("c")
    base = wid * _G_EPW

    def one(idx_hbm, out_hbm):
        def body(j, _):
            off = base + j * _G_K
            pltpu.sync_copy(idx_hbm.at[pl.ds(off, _G_K)], idx_v)
            pltpu.async_copy(nf_hbm.at[idx_v], rows_v, sem).wait()
            pltpu.sync_copy(rows_v, out_hbm.at[pl.ds(off, _G_K)])
            return _
        lax.fori_loop(0, _G_NCHUNK, body, 0)
        toff = base + _G_NCHUNK * _G_K
        pltpu.sync_copy(idx_hbm.at[pl.ds(toff, _G_TAIL)], idxt_v)
        pltpu.async_copy(nf_hbm.at[idxt_v], rowst_v, sem).wait()
        pltpu.sync_copy(rowst_v, out_hbm.at[pl.ds(toff, _G_TAIL)])

    one(snd_hbm, out_s_hbm)
    one(rcv_hbm, out_r_hbm)


@functools.lru_cache(maxsize=None)
def _sc_gather_kernel(width):
    return pl.kernel(
        _sc_gather_body,
        out_type=(jax.ShapeDtypeStruct((N_EDGES, width), jnp.float32),
                  jax.ShapeDtypeStruct((N_EDGES, width), jnp.float32)),
        mesh=_sc_mesh(),
        scratch_types=[
            pltpu.VMEM((_G_K,), jnp.int32),
            pltpu.VMEM((_G_K, width), jnp.float32),
            pltpu.VMEM((_G_TAIL,), jnp.int32),
            pltpu.VMEM((_G_TAIL, width), jnp.float32),
            pltpu.SemaphoreType.DMA,
        ],
    )


_S_H = D // NC
_S_EPW = N_EDGES // NS
_S_K = 128
_S_NCHUNK = _S_EPW // _S_K
_S_TAIL = _S_EPW - _S_NCHUNK * _S_K
_S_RPT = 624
_S_REM = N_NODES - NS * _S_RPT


def _sc_scatter_body(mij_hbm, rcv_hbm, zeros_hbm, out_hbm,
                     idx_v, rows_v, idxt_v, rowst_v, acc_sh, sem):
    c = lax.axis_index("c")
    s = lax.axis_index("s")
    col = c * _S_H
    pltpu.sync_copy(zeros_hbm.at[pl.ds(s * _S_RPT, _S_RPT)],
                    acc_sh.at[pl.ds(s * _S_RPT, _S_RPT)])
    @pl.when(s == 0)
    def _():
        pltpu.sync_copy(zeros_hbm.at[pl.ds(NS * _S_RPT, _S_REM)],
                        acc_sh.at[pl.ds(NS * _S_RPT, _S_REM)])
    plsc.subcore_barrier()

    base = s * _S_EPW

    def body(j, _):
        off = base + j * _S_K
        pltpu.sync_copy(rcv_hbm.at[pl.ds(off, _S_K)], idx_v)
        pltpu.sync_copy(mij_hbm.at[pl.ds(off, _S_K), pl.ds(col, _S_H)], rows_v)
        pltpu.sync_copy(rows_v, acc_sh.at[idx_v], add=True)
        return _
    lax.fori_loop(0, _S_NCHUNK, body, 0)
    toff = base + _S_NCHUNK * _S_K
    pltpu.sync_copy(rcv_hbm.at[pl.ds(toff, _S_TAIL)], idxt_v)
    pltpu.sync_copy(mij_hbm.at[pl.ds(toff, _S_TAIL), pl.ds(col, _S_H)], rowst_v)
    pltpu.sync_copy(rowst_v, acc_sh.at[idxt_v], add=True)

    plsc.subcore_barrier()
    pltpu.sync_copy(acc_sh.at[pl.ds(s * _S_RPT, _S_RPT)],
                    out_hbm.at[pl.ds(s * _S_RPT, _S_RPT), pl.ds(col, _S_H)])
    @pl.when(s == 0)
    def _():
        pltpu.sync_copy(acc_sh.at[pl.ds(NS * _S_RPT, _S_REM)],
                        out_hbm.at[pl.ds(NS * _S_RPT, _S_REM), pl.ds(col, _S_H)])


@functools.lru_cache(maxsize=None)
def _sc_scatter_kernel():
    return pl.kernel(
        _sc_scatter_body,
        out_type=jax.ShapeDtypeStruct((N_NODES, D), jnp.float32),
        mesh=_sc_mesh(),
        scratch_types=[
            pltpu.VMEM((_S_K,), jnp.int32),
            pltpu.VMEM((_S_K, _S_H), jnp.float32),
            pltpu.VMEM((_S_TAIL,), jnp.int32),
            pltpu.VMEM((_S_TAIL, _S_H), jnp.float32),
            pltpu.VMEM_SHARED((N_NODES, _S_H), jnp.float32),
            pltpu.SemaphoreType.DMA,
        ],
    )


_E_BLK = 1000


def _edge_mlp_body(sf_r, rf_r, ef_r, w1s_r, w1r_r, w1e_r, b1_r, w2_r, b2_r,
                   o_r):
    ef_bf = ef_r[...].astype(jnp.bfloat16)
    h = (jnp.dot(sf_r[...], w1s_r[...], preferred_element_type=jnp.float32)
         + jnp.dot(rf_r[...], w1r_r[...], preferred_element_type=jnp.float32)
         + jnp.dot(ef_bf, w1e_r[...], preferred_element_type=jnp.float32)
         + b1_r[...])
    h = jax.nn.gelu(h).astype(jnp.bfloat16)
    o_r[...] = (jnp.dot(h, w2_r[...], preferred_element_type=jnp.float32)
                + b2_r[...]) * 0.25


def _edge_mlp(sf, rf, ef, w1s, w1r, w1e, b1, w2, b2):
    nin = 3 * D
    grid = (N_EDGES // _E_BLK,)
    blk = lambda i: (i, 0)
    fixed = lambda i: (0, 0)
    return pl.pallas_call(
        _edge_mlp_body,
        grid=grid,
        in_specs=[
            pl.BlockSpec((_E_BLK, D), blk),
            pl.BlockSpec((_E_BLK, D), blk),
            pl.BlockSpec((_E_BLK, D), blk),
            pl.BlockSpec((D, nin), fixed),
            pl.BlockSpec((D, nin), fixed),
            pl.BlockSpec((D, nin), fixed),
            pl.BlockSpec((1, nin), fixed),
            pl.BlockSpec((nin, D), fixed),
            pl.BlockSpec((1, D), fixed),
        ],
        out_specs=pl.BlockSpec((_E_BLK, D), blk),
        out_shape=jax.ShapeDtypeStruct((N_EDGES, D), jnp.float32),
    )(sf, rf, ef, w1s, w1r, w1e, b1, w2, b2)


_N_BLK = 2000


def _node_mlp_body(mi_r, nf_r, w1m_r, w1x_r, b1_r, w2_r, b2_r, o_r):
    g = (jnp.dot(mi_r[...], w1m_r[...], preferred_element_type=jnp.float32)
         + jnp.dot(nf_r[...], w1x_r[...], preferred_element_type=jnp.float32)
         + b1_r[...])
    g = jax.nn.gelu(g)
    o_r[...] = (jnp.dot(g, w2_r[...], preferred_element_type=jnp.float32)
                + b2_r[...])


def _node_mlp(mi, nf, w1m, w1x, b1, w2, b2):
    nin = 2 * D
    grid = (N_NODES // _N_BLK,)
    blk = lambda i: (i, 0)
    fixed = lambda i: (0, 0)
    return pl.pallas_call(
        _node_mlp_body,
        grid=grid,
        in_specs=[
            pl.BlockSpec((_N_BLK, D), blk),
            pl.BlockSpec((_N_BLK, D), blk),
            pl.BlockSpec((D, nin), fixed),
            pl.BlockSpec((D, nin), fixed),
            pl.BlockSpec((1, nin), fixed),
            pl.BlockSpec((nin, D), fixed),
            pl.BlockSpec((1, D), fixed),
        ],
        out_specs=pl.BlockSpec((_N_BLK, D), blk),
        out_shape=jax.ShapeDtypeStruct((N_NODES, D), jnp.float32),
    )(mi, nf, w1m, w1x, b1, w2, b2)


def kernel(node_features, edge_features, senders, receivers,
           W1e, b1e, W2e, b2e, W1n, b1n, W2n, b2n):
    nf = node_features[0]
    ef = edge_features[0]
    snd = senders[0].astype(jnp.int32)
    rcv = receivers[0].astype(jnp.int32)

    nf_words = lax.bitcast_convert_type(
        node_features[0].astype(jnp.bfloat16).reshape(N_NODES, D // 2, 2),
        jnp.float32)
    sf_w, rf_w = _sc_gather_kernel(D // 2)(nf_words, snd, rcv)
    sf = lax.bitcast_convert_type(sf_w, jnp.bfloat16).reshape(N_EDGES, D)
    rf = lax.bitcast_convert_type(rf_w, jnp.bfloat16).reshape(N_EDGES, D)

    bf = jnp.bfloat16
    m_ij = _edge_mlp(sf, rf, ef,
                     W1e[:D].astype(bf), W1e[D:2 * D].astype(bf),
                     W1e[2 * D:].astype(bf),
                     b1e.reshape(1, -1), W2e.astype(bf), b2e.reshape(1, -1))

    zeros = jnp.zeros((N_NODES, _S_H), jnp.float32)
    m_i = _sc_scatter_kernel()(m_ij, rcv, zeros)

    out = _node_mlp(m_i, nf,
                    W1n[:D], W1n[D:],
                    b1n.reshape(1, -1), W2n, b2n.reshape(1, -1))
    return out[None]


# speedup vs baseline: 2.4188x; 2.4188x over previous
import functools

import jax
import jax.numpy as jnp
from jax import lax
from jax.experimental import pallas as pl
from jax.experimental.pallas import tpu as pltpu
from jax.experimental.pallas import tpu_sc as plsc

N_NODES = 10000
N_EDGES = 160000
D = 256

NC = 2
NS = 16
NW = NC * NS

@functools.lru_cache(maxsize=None)
def _sc_mesh():
    return plsc.VectorSubcoreMesh(core_axis_name="c", subcore_axis_name="s",
                                  num_cores=NC, num_subcores=NS)


_G_EPW = N_EDGES // NW
_G_K = 128
_G_NCHUNK = _G_EPW // _G_K
_G_TAIL = _G_EPW - _G_NCHUNK * _G_K


def _sc_gather_body(nf_hbm, snd_hbm, rcv_hbm, out_s_hbm, out_r_hbm,
                    idx_v, rows_v, idxt_v, rowst_v, sem):
    wid = lax.axis_index("s") * NC + lax.axis_index("c")
    base = wid * _G_EPW

    def one(idx_hbm, out_hbm):
        def body(j, _):
            off = base + j * _G_K
            pltpu.sync_copy(idx_hbm.at[pl.ds(off, _G_K)], idx_v)
            pltpu.async_copy(nf_hbm.at[idx_v], rows_v, sem).wait()
            pltpu.sync_copy(rows_v, out_hbm.at[pl.ds(off, _G_K)])
            return _
        lax.fori_loop(0, _G_NCHUNK, body, 0)
        toff = base + _G_NCHUNK * _G_K
        pltpu.sync_copy(idx_hbm.at[pl.ds(toff, _G_TAIL)], idxt_v)
        pltpu.async_copy(nf_hbm.at[idxt_v], rowst_v, sem).wait()
        pltpu.sync_copy(rowst_v, out_hbm.at[pl.ds(toff, _G_TAIL)])

    one(snd_hbm, out_s_hbm)
    one(rcv_hbm, out_r_hbm)


@functools.lru_cache(maxsize=None)
def _sc_gather_kernel(width):
    return pl.kernel(
        _sc_gather_body,
        out_type=(jax.ShapeDtypeStruct((N_EDGES, width), jnp.float32),
                  jax.ShapeDtypeStruct((N_EDGES, width), jnp.float32)),
        mesh=_sc_mesh(),
        scratch_types=[
            pltpu.VMEM((_G_K,), jnp.int32),
            pltpu.VMEM((_G_K, width), jnp.float32),
            pltpu.VMEM((_G_TAIL,), jnp.int32),
            pltpu.VMEM((_G_TAIL, width), jnp.float32),
            pltpu.SemaphoreType.DMA,
        ],
    )


_S_H = D // NC
_S_EPW = N_EDGES // NS
_S_K = 128
_S_NCHUNK = _S_EPW // _S_K
_S_TAIL = _S_EPW - _S_NCHUNK * _S_K
_S_RPT = 624
_S_REM = N_NODES - NS * _S_RPT


def _sc_scatter_body(mij_hbm, rcv_hbm, zeros_hbm, out_hbm,
                     idx_v, rows_v, idxt_v, rowst_v, acc_sh, sem):
    c = lax.axis_index("c")
    s = lax.axis_index("s")
    col = c * _S_H
    pltpu.sync_copy(zeros_hbm.at[pl.ds(s * _S_RPT, _S_RPT)],
                    acc_sh.at[pl.ds(s * _S_RPT, _S_RPT)])
    @pl.when(s == 0)
    def _():
        pltpu.sync_copy(zeros_hbm.at[pl.ds(NS * _S_RPT, _S_REM)],
                        acc_sh.at[pl.ds(NS * _S_RPT, _S_REM)])
    plsc.subcore_barrier()

    base = s * _S_EPW

    def body(j, _):
        off = base + j * _S_K
        pltpu.sync_copy(rcv_hbm.at[pl.ds(off, _S_K)], idx_v)
        pltpu.sync_copy(mij_hbm.at[pl.ds(off, _S_K), pl.ds(col, _S_H)], rows_v)
        pltpu.sync_copy(rows_v, acc_sh.at[idx_v], add=True)
        return _
    lax.fori_loop(0, _S_NCHUNK, body, 0)
    toff = base + _S_NCHUNK * _S_K
    pltpu.sync_copy(rcv_hbm.at[pl.ds(toff, _S_TAIL)], idxt_v)
    pltpu.sync_copy(mij_hbm.at[pl.ds(toff, _S_TAIL), pl.ds(col, _S_H)], rowst_v)
    pltpu.sync_copy(rowst_v, acc_sh.at[idxt_v], add=True)

    plsc.subcore_barrier()
    pltpu.sync_copy(acc_sh.at[pl.ds(s * _S_RPT, _S_RPT)],
                    out_hbm.at[pl.ds(s * _S_RPT, _S_RPT), pl.ds(col, _S_H)])
    @pl.when(s == 0)
    def _():
        pltpu.sync_copy(acc_sh.at[pl.ds(NS * _S_RPT, _S_REM)],
                        out_hbm.at[pl.ds(NS * _S_RPT, _S_REM), pl.ds(col, _S_H)])


@functools.lru_cache(maxsize=None)
def _sc_scatter_kernel():
    return pl.kernel(
        _sc_scatter_body,
        out_type=jax.ShapeDtypeStruct((N_NODES, D), jnp.float32),
        mesh=_sc_mesh(),
        scratch_types=[
            pltpu.VMEM((_S_K,), jnp.int32),
            pltpu.VMEM((_S_K, _S_H), jnp.float32),
            pltpu.VMEM((_S_TAIL,), jnp.int32),
            pltpu.VMEM((_S_TAIL, _S_H), jnp.float32),
            pltpu.VMEM_SHARED((N_NODES, _S_H), jnp.float32),
            pltpu.SemaphoreType.DMA,
        ],
    )


_P_BLK = 2000
_HW = D // 2


def _pack_body(nf_r, o_r):
    x = nf_r[...]
    lo = x[:, :_HW].astype(jnp.bfloat16).astype(jnp.float32)
    hi = x[:, _HW:].astype(jnp.bfloat16).astype(jnp.float32)
    lo_u = lax.bitcast_convert_type(lo, jnp.uint32)
    hi_u = lax.bitcast_convert_type(hi, jnp.uint32)
    w = hi_u | (lo_u >> 16)
    o_r[...] = lax.bitcast_convert_type(w, jnp.float32)


def _pack(nf):
    grid = (N_NODES // _P_BLK,)
    return pl.pallas_call(
        _pack_body,
        grid=grid,
        in_specs=[pl.BlockSpec((_P_BLK, D), lambda i: (i, 0))],
        out_specs=pl.BlockSpec((_P_BLK, _HW), lambda i: (i, 0)),
        out_shape=jax.ShapeDtypeStruct((N_NODES, _HW), jnp.float32),
    )(nf)


def _unpack(w_f32):
    u = lax.bitcast_convert_type(w_f32, jnp.uint32)
    hi = lax.bitcast_convert_type(u & jnp.uint32(0xFFFF0000), jnp.float32)
    lo = lax.bitcast_convert_type(u << 16, jnp.float32)
    return jnp.concatenate([lo, hi], axis=1).astype(jnp.bfloat16)


_E_BLK = 1000


def _edge_mlp_body(sf_r, rf_r, ef_r, w1s_r, w1r_r, w1e_r, b1_r, w2_r, b2_r,
                   o_r):
    sfx = _unpack(sf_r[...])
    rfx = _unpack(rf_r[...])
    ef_bf = ef_r[...].astype(jnp.bfloat16)
    h = (jnp.dot(sfx, w1s_r[...], preferred_element_type=jnp.float32)
         + jnp.dot(rfx, w1r_r[...], preferred_element_type=jnp.float32)
         + jnp.dot(ef_bf, w1e_r[...], preferred_element_type=jnp.float32)
         + b1_r[...])
    h = jax.nn.gelu(h).astype(jnp.bfloat16)
    o_r[...] = (jnp.dot(h, w2_r[...], preferred_element_type=jnp.float32)
                + b2_r[...]) * 0.25


def _edge_mlp(sf, rf, ef, w1s, w1r, w1e, b1, w2, b2):
    nin = 3 * D
    grid = (N_EDGES // _E_BLK,)
    blk = lambda i: (i, 0)
    fixed = lambda i: (0, 0)
    return pl.pallas_call(
        _edge_mlp_body,
        grid=grid,
        in_specs=[
            pl.BlockSpec((_E_BLK, _HW), blk),
            pl.BlockSpec((_E_BLK, _HW), blk),
            pl.BlockSpec((_E_BLK, D), blk),
            pl.BlockSpec((D, nin), fixed),
            pl.BlockSpec((D, nin), fixed),
            pl.BlockSpec((D, nin), fixed),
            pl.BlockSpec((1, nin), fixed),
            pl.BlockSpec((nin, D), fixed),
            pl.BlockSpec((1, D), fixed),
        ],
        out_specs=pl.BlockSpec((_E_BLK, D), blk),
        out_shape=jax.ShapeDtypeStruct((N_EDGES, D), jnp.float32),
    )(sf, rf, ef, w1s, w1r, w1e, b1, w2, b2)


_N_BLK = 2000


def _node_mlp_body(mi_r, nf_r, w1m_r, w1x_r, b1_r, w2_r, b2_r, o_r):
    g = (jnp.dot(mi_r[...], w1m_r[...], preferred_element_type=jnp.float32)
         + jnp.dot(nf_r[...], w1x_r[...], preferred_element_type=jnp.float32)
         + b1_r[...])
    g = jax.nn.gelu(g)
    o_r[...] = (jnp.dot(g, w2_r[...], preferred_element_type=jnp.float32)
                + b2_r[...])


def _node_mlp(mi, nf, w1m, w1x, b1, w2, b2):
    nin = 2 * D
    grid = (N_NODES // _N_BLK,)
    blk = lambda i: (i, 0)
    fixed = lambda i: (0, 0)
    return pl.pallas_call(
        _node_mlp_body,
        grid=grid,
        in_specs=[
            pl.BlockSpec((_N_BLK, D), blk),
            pl.BlockSpec((_N_BLK, D), blk),
            pl.BlockSpec((D, nin), fixed),
            pl.BlockSpec((D, nin), fixed),
            pl.BlockSpec((1, nin), fixed),
            pl.BlockSpec((nin, D), fixed),
            pl.BlockSpec((1, D), fixed),
        ],
        out_specs=pl.BlockSpec((_N_BLK, D), blk),
        out_shape=jax.ShapeDtypeStruct((N_NODES, D), jnp.float32),
    )(mi, nf, w1m, w1x, b1, w2, b2)


def kernel(node_features, edge_features, senders, receivers,
           W1e, b1e, W2e, b2e, W1n, b1n, W2n, b2n):
    nf = node_features[0]
    ef = edge_features[0]
    snd = senders[0].astype(jnp.int32)
    rcv = receivers[0].astype(jnp.int32)

    nf_words = _pack(nf)
    sf_w, rf_w = _sc_gather_kernel(_HW)(nf_words, snd, rcv)

    bf = jnp.bfloat16
    m_ij = _edge_mlp(sf_w, rf_w, ef,
                     W1e[:D].astype(bf), W1e[D:2 * D].astype(bf),
                     W1e[2 * D:].astype(bf),
                     b1e.reshape(1, -1), W2e.astype(bf), b2e.reshape(1, -1))

    zeros = jnp.zeros((N_NODES, _S_H), jnp.float32)
    m_i = _sc_scatter_kernel()(m_ij, rcv, zeros)

    out = _node_mlp(m_i, nf,
                    W1n[:D], W1n[D:],
                    b1n.reshape(1, -1), W2n, b2n.reshape(1, -1))
    return out[None]


# speedup vs baseline: 3.1805x; 1.3149x over previous
import functools

import jax
import jax.numpy as jnp
from jax import lax
from jax.experimental import pallas as pl
from jax.experimental.pallas import tpu as pltpu
from jax.experimental.pallas import tpu_sc as plsc

N_NODES = 10000
N_EDGES = 160000
D = 256

NC = 2
NS = 16
NW = NC * NS
_NSLAB = 5

@functools.lru_cache(maxsize=None)
def _sc_mesh():
    return plsc.VectorSubcoreMesh(core_axis_name="c", subcore_axis_name="s",
                                  num_cores=NC, num_subcores=NS)


_G_K = 128


@functools.lru_cache(maxsize=None)
def _sc_gather_kernel(width, slab_base, slab_edges):
    epw = slab_edges // NW
    nchunk = epw // _G_K
    tail = epw - nchunk * _G_K

    def body(nf_hbm, snd_hbm, rcv_hbm, out_s_hbm, out_r_hbm,
             idx_v, rows_v, idxt_v, rowst_v, sem):
        wid = lax.axis_index("s") * NC + lax.axis_index("c")
        loc = wid * epw

        def one(idx_hbm, out_hbm):
            def chunk(j, _):
                off = loc + j * _G_K
                pltpu.sync_copy(idx_hbm.at[pl.ds(slab_base + off, _G_K)],
                                idx_v)
                pltpu.async_copy(nf_hbm.at[idx_v], rows_v, sem).wait()
                pltpu.sync_copy(rows_v, out_hbm.at[pl.ds(off, _G_K)])
                return _
            lax.fori_loop(0, nchunk, chunk, 0)
            if tail:
                toff = loc + nchunk * _G_K
                pltpu.sync_copy(idx_hbm.at[pl.ds(slab_base + toff, tail)],
                                idxt_v)
                pltpu.async_copy(nf_hbm.at[idxt_v], rowst_v, sem).wait()
                pltpu.sync_copy(rowst_v, out_hbm.at[pl.ds(toff, tail)])

        one(snd_hbm, out_s_hbm)
        one(rcv_hbm, out_r_hbm)

    return pl.kernel(
        body,
        out_type=(jax.ShapeDtypeStruct((slab_edges, width), jnp.float32),
                  jax.ShapeDtypeStruct((slab_edges, width), jnp.float32)),
        mesh=_sc_mesh(),
        scratch_types=[
            pltpu.VMEM((_G_K,), jnp.int32),
            pltpu.VMEM((_G_K, width), jnp.float32),
            pltpu.VMEM((max(tail, 8),), jnp.int32),
            pltpu.VMEM((max(tail, 8), width), jnp.float32),
            pltpu.SemaphoreType.DMA,
        ],
    )


_S_H = D // NC
_S_K = 128
_S_RPT = 624
_S_REM = N_NODES - NS * _S_RPT


@functools.lru_cache(maxsize=None)
def _sc_scatter_kernel(slab_base, slab_edges):
    ept = slab_edges // NS
    nchunk = ept // _S_K
    tail = ept - nchunk * _S_K

    def body(mij_hbm, rcv_hbm, zeros_hbm, out_hbm,
             idx_v, rows_v, idxt_v, rowst_v, acc_sh, sem):
        c = lax.axis_index("c")
        s = lax.axis_index("s")
        col = c * _S_H
        pltpu.sync_copy(zeros_hbm.at[pl.ds(s * _S_RPT, _S_RPT)],
                        acc_sh.at[pl.ds(s * _S_RPT, _S_RPT)])
        @pl.when(s == 0)
        def _():
            pltpu.sync_copy(zeros_hbm.at[pl.ds(NS * _S_RPT, _S_REM)],
                            acc_sh.at[pl.ds(NS * _S_RPT, _S_REM)])
        plsc.subcore_barrier()

        loc = s * ept

        def chunk(j, _):
            off = loc + j * _S_K
            pltpu.sync_copy(rcv_hbm.at[pl.ds(slab_base + off, _S_K)], idx_v)
            pltpu.sync_copy(mij_hbm.at[pl.ds(off, _S_K), pl.ds(col, _S_H)],
                            rows_v)
            pltpu.sync_copy(rows_v, acc_sh.at[idx_v], add=True)
            return _
        lax.fori_loop(0, nchunk, chunk, 0)
        if tail:
            toff = loc + nchunk * _S_K
            pltpu.sync_copy(rcv_hbm.at[pl.ds(slab_base + toff, tail)], idxt_v)
            pltpu.sync_copy(mij_hbm.at[pl.ds(toff, tail), pl.ds(col, _S_H)],
                            rowst_v)
            pltpu.sync_copy(rowst_v, acc_sh.at[idxt_v], add=True)

        plsc.subcore_barrier()
        pltpu.sync_copy(acc_sh.at[pl.ds(s * _S_RPT, _S_RPT)],
                        out_hbm.at[pl.ds(s * _S_RPT, _S_RPT), pl.ds(col, _S_H)])
        @pl.when(s == 0)
        def _():
            pltpu.sync_copy(
                acc_sh.at[pl.ds(NS * _S_RPT, _S_REM)],
                out_hbm.at[pl.ds(NS * _S_RPT, _S_REM), pl.ds(col, _S_H)])

    return pl.kernel(
        body,
        out_type=jax.ShapeDtypeStruct((N_NODES, D), jnp.float32),
        mesh=_sc_mesh(),
        scratch_types=[
            pltpu.VMEM((_S_K,), jnp.int32),
            pltpu.VMEM((_S_K, _S_H), jnp.float32),
            pltpu.VMEM((max(tail, 8),), jnp.int32),
            pltpu.VMEM((max(tail, 8), _S_H), jnp.float32),
            pltpu.VMEM_SHARED((N_NODES, _S_H), jnp.float32),
            pltpu.SemaphoreType.DMA,
        ],
    )


_P_BLK = 2000
_HW = D // 2


def _pack_body(nf_r, o_r):
    x = nf_r[...]
    lo = x[:, :_HW].astype(jnp.bfloat16).astype(jnp.float32)
    hi = x[:, _HW:].astype(jnp.bfloat16).astype(jnp.float32)
    lo_u = lax.bitcast_convert_type(lo, jnp.uint32)
    hi_u = lax.bitcast_convert_type(hi, jnp.uint32)
    w = hi_u | (lo_u >> 16)
    o_r[...] = lax.bitcast_convert_type(w, jnp.float32)


def _pack(nf):
    grid = (N_NODES // _P_BLK,)
    return pl.pallas_call(
        _pack_body,
        grid=grid,
        in_specs=[pl.BlockSpec((_P_BLK, D), lambda i: (i, 0))],
        out_specs=pl.BlockSpec((_P_BLK, _HW), lambda i: (i, 0)),
        out_shape=jax.ShapeDtypeStruct((N_NODES, _HW), jnp.float32),
    )(nf)


def _unpack(w_f32):
    u = lax.bitcast_convert_type(w_f32, jnp.uint32)
    hi = lax.bitcast_convert_type(u & jnp.uint32(0xFFFF0000), jnp.float32)
    lo = lax.bitcast_convert_type(u << 16, jnp.float32)
    return jnp.concatenate([lo, hi], axis=1).astype(jnp.bfloat16)


_E_BLK = 1000


def _edge_mlp_body(sf_r, rf_r, ef_r, w1s_r, w1r_r, w1e_r, b1_r, w2_r, b2_r,
                   o_r):
    sfx = _unpack(sf_r[...])
    rfx = _unpack(rf_r[...])
    ef_bf = ef_r[...].astype(jnp.bfloat16)
    h = (jnp.dot(sfx, w1s_r[...], preferred_element_type=jnp.float32)
         + jnp.dot(rfx, w1r_r[...], preferred_element_type=jnp.float32)
         + jnp.dot(ef_bf, w1e_r[...], preferred_element_type=jnp.float32)
         + b1_r[...])
    h = jax.nn.gelu(h).astype(jnp.bfloat16)
    o_r[...] = (jnp.dot(h, w2_r[...], preferred_element_type=jnp.float32)
                + b2_r[...]) * 0.25


def _edge_mlp(sf, rf, ef, w1s, w1r, w1e, b1, w2, b2, slab_base, slab_edges):
    nin = 3 * D
    grid = (slab_edges // _E_BLK,)
    base_blk = slab_base // _E_BLK
    blk = lambda i: (i, 0)
    efblk = lambda i: (base_blk + i, 0)
    fixed = lambda i: (0, 0)
    return pl.pallas_call(
        _edge_mlp_body,
        grid=grid,
        in_specs=[
            pl.BlockSpec((_E_BLK, _HW), blk),
            pl.BlockSpec((_E_BLK, _HW), blk),
            pl.BlockSpec((_E_BLK, D), efblk),
            pl.BlockSpec((D, nin), fixed),
            pl.BlockSpec((D, nin), fixed),
            pl.BlockSpec((D, nin), fixed),
            pl.BlockSpec((1, nin), fixed),
            pl.BlockSpec((nin, D), fixed),
            pl.BlockSpec((1, D), fixed),
        ],
        out_specs=pl.BlockSpec((_E_BLK, D), blk),
        out_shape=jax.ShapeDtypeStruct((slab_edges, D), jnp.float32),
    )(sf, rf, ef, w1s, w1r, w1e, b1, w2, b2)


_N_BLK = 2000


def _node_mlp_body(*refs):
    o_r = refs[-1]
    parts = refs[:-7]
    nf_r, w1m_r, w1x_r, b1_r, w2_r, b2_r = refs[-7:-1]
    mi = parts[0][...]
    for p in parts[1:]:
        mi = mi + p[...]
    g = (jnp.dot(mi, w1m_r[...], preferred_element_type=jnp.float32)
         + jnp.dot(nf_r[...], w1x_r[...], preferred_element_type=jnp.float32)
         + b1_r[...])
    g = jax.nn.gelu(g)
    o_r[...] = (jnp.dot(g, w2_r[...], preferred_element_type=jnp.float32)
                + b2_r[...])


def _node_mlp(mi_parts, nf, w1m, w1x, b1, w2, b2):
    nin = 2 * D
    grid = (N_NODES // _N_BLK,)
    blk = lambda i: (i, 0)
    fixed = lambda i: (0, 0)
    return pl.pallas_call(
        _node_mlp_body,
        grid=grid,
        in_specs=[pl.BlockSpec((_N_BLK, D), blk)] * len(mi_parts) + [
            pl.BlockSpec((_N_BLK, D), blk),
            pl.BlockSpec((D, nin), fixed),
            pl.BlockSpec((D, nin), fixed),
            pl.BlockSpec((1, nin), fixed),
            pl.BlockSpec((nin, D), fixed),
            pl.BlockSpec((1, D), fixed),
        ],
        out_specs=pl.BlockSpec((_N_BLK, D), blk),
        out_shape=jax.ShapeDtypeStruct((N_NODES, D), jnp.float32),
    )(*mi_parts, nf, w1m, w1x, b1, w2, b2)


def kernel(node_features, edge_features, senders, receivers,
           W1e, b1e, W2e, b2e, W1n, b1n, W2n, b2n):
    nf = node_features[0]
    ef = edge_features[0]
    snd = senders[0].astype(jnp.int32)
    rcv = receivers[0].astype(jnp.int32)

    nf_words = _pack(nf)
    zeros = jnp.zeros((N_NODES, _S_H), jnp.float32)
    bf = jnp.bfloat16
    w1s, w1r, w1x = (W1e[:D].astype(bf), W1e[D:2 * D].astype(bf),
                     W1e[2 * D:].astype(bf))
    b1 = b1e.reshape(1, -1)
    w2 = W2e.astype(bf)
    b2 = b2e.reshape(1, -1)

    slab = N_EDGES // _NSLAB
    parts = []
    for i in range(_NSLAB):
        base = i * slab
        sf_w, rf_w = _sc_gather_kernel(_HW, base, slab)(nf_words, snd, rcv)
        m_ij = _edge_mlp(sf_w, rf_w, ef, w1s, w1r, w1x, b1, w2, b2,
                         base, slab)
        parts.append(_sc_scatter_kernel(base, slab)(m_ij, rcv, zeros))

    out = _node_mlp(parts, nf,
                    W1n[:D], W1n[D:],
                    b1n.reshape(1, -1), W2n, b2n.reshape(1, -1))
    return out[None]


# speedup vs baseline: 3.3731x; 1.0605x over previous
import functools

import jax
import jax.numpy as jnp
from jax import lax
from jax.experimental import pallas as pl
from jax.experimental.pallas import tpu as pltpu
from jax.experimental.pallas import tpu_sc as plsc

N_NODES = 10000
N_EDGES = 160000
D = 256

NC = 2
NS = 16
NW = NC * NS
_NSLAB = 5
_SGROUPS = ((0, 1), (2, 3), (4,))

@functools.lru_cache(maxsize=None)
def _sc_mesh():
    return plsc.VectorSubcoreMesh(core_axis_name="c", subcore_axis_name="s",
                                  num_cores=NC, num_subcores=NS)


_G_K = 128


@functools.lru_cache(maxsize=None)
def _sc_gather_kernel(width, slab_base, slab_edges):
    epw = slab_edges // NW
    nchunk = epw // _G_K
    tail = epw - nchunk * _G_K

    def body(nf_hbm, snd_hbm, rcv_hbm, out_s_hbm, out_r_hbm,
             idx_v, rows_v, idxt_v, rowst_v, sem):
        wid = lax.axis_index("s") * NC + lax.axis_index("c")
        loc = wid * epw

        def one(idx_hbm, out_hbm):
            def chunk(j, _):
                off = loc + j * _G_K
                pltpu.sync_copy(idx_hbm.at[pl.ds(slab_base + off, _G_K)],
                                idx_v)
                pltpu.async_copy(nf_hbm.at[idx_v], rows_v, sem).wait()
                pltpu.sync_copy(rows_v, out_hbm.at[pl.ds(off, _G_K)])
                return _
            lax.fori_loop(0, nchunk, chunk, 0)
            if tail:
                toff = loc + nchunk * _G_K
                pltpu.sync_copy(idx_hbm.at[pl.ds(slab_base + toff, tail)],
                                idxt_v)
                pltpu.async_copy(nf_hbm.at[idxt_v], rowst_v, sem).wait()
                pltpu.sync_copy(rowst_v, out_hbm.at[pl.ds(toff, tail)])

        one(snd_hbm, out_s_hbm)
        one(rcv_hbm, out_r_hbm)

    return pl.kernel(
        body,
        out_type=(jax.ShapeDtypeStruct((slab_edges, width), jnp.float32),
                  jax.ShapeDtypeStruct((slab_edges, width), jnp.float32)),
        mesh=_sc_mesh(),
        scratch_types=[
            pltpu.VMEM((_G_K,), jnp.int32),
            pltpu.VMEM((_G_K, width), jnp.float32),
            pltpu.VMEM((max(tail, 8),), jnp.int32),
            pltpu.VMEM((max(tail, 8), width), jnp.float32),
            pltpu.SemaphoreType.DMA,
        ],
    )


_S_H = D // NC
_S_K = 128
_S_RPT = 624
_S_REM = N_NODES - NS * _S_RPT
_Z_R = 104


@functools.lru_cache(maxsize=None)
def _sc_scatter_kernel(bases, slab_edges):
    ng = len(bases)
    tpg = NS // ng
    ept = slab_edges // tpg
    nchunk = ept // _S_K
    tail = ept - nchunk * _S_K

    def body(*refs):
        mij_hbms = refs[:ng]
        rcv_hbm, zeros_hbm, out_hbm = refs[ng:ng + 3]
        idx_v, rows_v, idxt_v, rowst_v, zbuf, acc_sh, sem = refs[ng + 3:]
        c = lax.axis_index("c")
        s = lax.axis_index("s")
        col = c * _S_H
        pltpu.sync_copy(zeros_hbm, zbuf)
        for r in range(_S_RPT // _Z_R):
            pltpu.sync_copy(zbuf, acc_sh.at[pl.ds(s * _S_RPT + r * _Z_R,
                                                  _Z_R)])
        @pl.when(s == 0)
        def _():
            pltpu.sync_copy(zbuf.at[pl.ds(0, _S_REM)],
                            acc_sh.at[pl.ds(NS * _S_RPT, _S_REM)])
        plsc.subcore_barrier()

        def scan(mij_hbm, ebase, loc):
            def chunk(j, _):
                off = loc + j * _S_K
                pltpu.sync_copy(rcv_hbm.at[pl.ds(ebase + off, _S_K)], idx_v)
                pltpu.sync_copy(mij_hbm.at[pl.ds(off, _S_K), pl.ds(col, _S_H)],
                                rows_v)
                pltpu.sync_copy(rows_v, acc_sh.at[idx_v], add=True)
                return _
            lax.fori_loop(0, nchunk, chunk, 0)
            if tail:
                toff = loc + nchunk * _S_K
                pltpu.sync_copy(rcv_hbm.at[pl.ds(ebase + toff, tail)], idxt_v)
                pltpu.sync_copy(mij_hbm.at[pl.ds(toff, tail), pl.ds(col, _S_H)],
                                rowst_v)
                pltpu.sync_copy(rowst_v, acc_sh.at[idxt_v], add=True)

        for gi in range(ng):
            @pl.when((s >= gi * tpg) & (s < (gi + 1) * tpg))
            def _(gi=gi):
                scan(mij_hbms[gi], bases[gi], (s - gi * tpg) * ept)

        plsc.subcore_barrier()
        pltpu.sync_copy(acc_sh.at[pl.ds(s * _S_RPT, _S_RPT)],
                        out_hbm.at[pl.ds(s * _S_RPT, _S_RPT), pl.ds(col, _S_H)])
        @pl.when(s == 0)
        def _():
            pltpu.sync_copy(
                acc_sh.at[pl.ds(NS * _S_RPT, _S_REM)],
                out_hbm.at[pl.ds(NS * _S_RPT, _S_REM), pl.ds(col, _S_H)])

    return pl.kernel(
        body,
        out_type=jax.ShapeDtypeStruct((N_NODES, D), jnp.float32),
        mesh=_sc_mesh(),
        scratch_types=[
            pltpu.VMEM((_S_K,), jnp.int32),
            pltpu.VMEM((_S_K, _S_H), jnp.float32),
            pltpu.VMEM((max(tail, 8),), jnp.int32),
            pltpu.VMEM((max(tail, 8), _S_H), jnp.float32),
            pltpu.VMEM((_Z_R, _S_H), jnp.float32),
            pltpu.VMEM_SHARED((N_NODES, _S_H), jnp.float32),
            pltpu.SemaphoreType.DMA,
        ],
    )


_P_BLK = 2000
_HW = D // 2


def _pack_body(nf_r, o_r):
    x = nf_r[...]
    lo = x[:, :_HW].astype(jnp.bfloat16).astype(jnp.float32)
    hi = x[:, _HW:].astype(jnp.bfloat16).astype(jnp.float32)
    lo_u = lax.bitcast_convert_type(lo, jnp.uint32)
    hi_u = lax.bitcast_convert_type(hi, jnp.uint32)
    w = hi_u | (lo_u >> 16)
    o_r[...] = lax.bitcast_convert_type(w, jnp.float32)


def _pack(nf):
    grid = (N_NODES // _P_BLK,)
    return pl.pallas_call(
        _pack_body,
        grid=grid,
        in_specs=[pl.BlockSpec((_P_BLK, D), lambda i: (i, 0))],
        out_specs=pl.BlockSpec((_P_BLK, _HW), lambda i: (i, 0)),
        out_shape=jax.ShapeDtypeStruct((N_NODES, _HW), jnp.float32),
    )(nf)


def _unpack(w_f32):
    u = lax.bitcast_convert_type(w_f32, jnp.uint32)
    hi = lax.bitcast_convert_type(u & jnp.uint32(0xFFFF0000), jnp.float32)
    lo = lax.bitcast_convert_type(u << 16, jnp.float32)
    return jnp.concatenate([lo, hi], axis=1).astype(jnp.bfloat16)


_E_BLK = 1000


def _edge_mlp_body(sf_r, rf_r, ef_r, w1s_r, w1r_r, w1e_r, b1_r, w2_r, b2_r,
                   o_r):
    sfx = _unpack(sf_r[...])
    rfx = _unpack(rf_r[...])
    ef_bf = ef_r[...].astype(jnp.bfloat16)
    h = (jnp.dot(sfx, w1s_r[...], preferred_element_type=jnp.float32)
         + jnp.dot(rfx, w1r_r[...], preferred_element_type=jnp.float32)
         + jnp.dot(ef_bf, w1e_r[...], preferred_element_type=jnp.float32)
         + b1_r[...])
    h = jax.nn.gelu(h).astype(jnp.bfloat16)
    o_r[...] = (jnp.dot(h, w2_r[...], preferred_element_type=jnp.float32)
                + b2_r[...]) * 0.25


def _edge_mlp(sf, rf, ef, w1s, w1r, w1e, b1, w2, b2, slab_base, slab_edges):
    nin = 3 * D
    grid = (slab_edges // _E_BLK,)
    base_blk = slab_base // _E_BLK
    blk = lambda i: (i, 0)
    efblk = lambda i: (base_blk + i, 0)
    fixed = lambda i: (0, 0)
    return pl.pallas_call(
        _edge_mlp_body,
        grid=grid,
        in_specs=[
            pl.BlockSpec((_E_BLK, _HW), blk),
            pl.BlockSpec((_E_BLK, _HW), blk),
            pl.BlockSpec((_E_BLK, D), efblk),
            pl.BlockSpec((D, nin), fixed),
            pl.BlockSpec((D, nin), fixed),
            pl.BlockSpec((D, nin), fixed),
            pl.BlockSpec((1, nin), fixed),
            pl.BlockSpec((nin, D), fixed),
            pl.BlockSpec((1, D), fixed),
        ],
        out_specs=pl.BlockSpec((_E_BLK, D), blk),
        out_shape=jax.ShapeDtypeStruct((slab_edges, D), jnp.float32),
    )(sf, rf, ef, w1s, w1r, w1e, b1, w2, b2)


_N_BLK = 2000


def _node_mlp_body(*refs):
    o_r = refs[-1]
    parts = refs[:-7]
    nf_r, w1m_r, w1x_r, b1_r, w2_r, b2_r = refs[-7:-1]
    mi = parts[0][...]
    for p in parts[1:]:
        mi = mi + p[...]
    g = (jnp.dot(mi, w1m_r[...], preferred_element_type=jnp.float32)
         + jnp.dot(nf_r[...], w1x_r[...], preferred_element_type=jnp.float32)
         + b1_r[...])
    g = jax.nn.gelu(g)
    o_r[...] = (jnp.dot(g, w2_r[...], preferred_element_type=jnp.float32)
                + b2_r[...])


def _node_mlp(mi_parts, nf, w1m, w1x, b1, w2, b2):
    nin = 2 * D
    grid = (N_NODES // _N_BLK,)
    blk = lambda i: (i, 0)
    fixed = lambda i: (0, 0)
    return pl.pallas_call(
        _node_mlp_body,
        grid=grid,
        in_specs=[pl.BlockSpec((_N_BLK, D), blk)] * len(mi_parts) + [
            pl.BlockSpec((_N_BLK, D), blk),
            pl.BlockSpec((D, nin), fixed),
            pl.BlockSpec((D, nin), fixed),
            pl.BlockSpec((1, nin), fixed),
            pl.BlockSpec((nin, D), fixed),
            pl.BlockSpec((1, D), fixed),
        ],
        out_specs=pl.BlockSpec((_N_BLK, D), blk),
        out_shape=jax.ShapeDtypeStruct((N_NODES, D), jnp.float32),
    )(*mi_parts, nf, w1m, w1x, b1, w2, b2)


def kernel(node_features, edge_features, senders, receivers,
           W1e, b1e, W2e, b2e, W1n, b1n, W2n, b2n):
    nf = node_features[0]
    ef = edge_features[0]
    snd = senders[0].astype(jnp.int32)
    rcv = receivers[0].astype(jnp.int32)

    nf_words = _pack(nf)
    zeros = jnp.zeros((_Z_R, _S_H), jnp.float32)
    bf = jnp.bfloat16
    w1s, w1r, w1x = (W1e[:D].astype(bf), W1e[D:2 * D].astype(bf),
                     W1e[2 * D:].astype(bf))
    b1 = b1e.reshape(1, -1)
    w2 = W2e.astype(bf)
    b2 = b2e.reshape(1, -1)

    slab = N_EDGES // _NSLAB
    m_ijs = []
    for i in range(_NSLAB):
        base = i * slab
        sf_w, rf_w = _sc_gather_kernel(_HW, base, slab)(nf_words, snd, rcv)
        m_ijs.append(_edge_mlp(sf_w, rf_w, ef, w1s, w1r, w1x, b1, w2, b2,
                               base, slab))

    parts = []
    for grp in _SGROUPS:
        bases = tuple(i * slab for i in grp)
        parts.append(_sc_scatter_kernel(bases, slab)(
            *[m_ijs[i] for i in grp], rcv, zeros))

    out = _node_mlp(parts, nf,
                    W1n[:D], W1n[D:],
                    b1n.reshape(1, -1), W2n, b2n.reshape(1, -1))
    return out[None]


# speedup vs baseline: 3.5370x; 1.0486x over previous
import functools

import jax
import jax.numpy as jnp
from jax import lax
from jax.experimental import pallas as pl
from jax.experimental.pallas import tpu as pltpu
from jax.experimental.pallas import tpu_sc as plsc

N_NODES = 10000
N_EDGES = 160000
D = 256

NC = 2
NS = 16
NW = NC * NS
_NSLAB = 5
_SGROUPS = ((0, 1), (2, 3), (4,))

@functools.lru_cache(maxsize=None)
def _sc_mesh():
    return plsc.VectorSubcoreMesh(core_axis_name="c", subcore_axis_name="s",
                                  num_cores=NC, num_subcores=NS)


_G_K = 128


@functools.lru_cache(maxsize=None)
def _sc_gather_kernel(width, slab_base, slab_edges):
    epw = slab_edges // NW
    nchunk = epw // _G_K
    tail = epw - nchunk * _G_K

    def body(nf_hbm, snd_hbm, rcv_hbm, out_s_hbm, out_r_hbm,
             idx0, idx1, rows0, rows1, idxt, rowst,
             sem0, sem1, semt, wsem0, wsem1, wsemt):
        idxs = (idx0, idx1)
        rows = (rows0, rows1)
        sems = (sem0, sem1)
        wsems = (wsem0, wsem1)
        wid = lax.axis_index("s") * NC + lax.axis_index("c")
        loc = wid * epw

        def one(idx_hbm, out_hbm):
            cps = [None, None]
            wcps = [None, None]
            for j in range(nchunk):
                b = j & 1
                if wcps[b] is not None:
                    wcps[b].wait()
                    wcps[b] = None
                pltpu.sync_copy(
                    idx_hbm.at[pl.ds(slab_base + loc + j * _G_K, _G_K)],
                    idxs[b])
                cps[b] = pltpu.async_copy(nf_hbm.at[idxs[b]], rows[b],
                                          sems[b])
                if j:
                    cps[1 - b].wait()
                    wcps[1 - b] = pltpu.async_copy(
                        rows[1 - b],
                        out_hbm.at[pl.ds(loc + (j - 1) * _G_K, _G_K)],
                        wsems[1 - b])
            cpt = wct = None
            if tail:
                toff = loc + nchunk * _G_K
                pltpu.sync_copy(idx_hbm.at[pl.ds(slab_base + toff, tail)],
                                idxt)
                cpt = pltpu.async_copy(nf_hbm.at[idxt], rowst, semt)
            bl = (nchunk - 1) & 1
            cps[bl].wait()
            wcps[bl] = pltpu.async_copy(
                rows[bl], out_hbm.at[pl.ds(loc + (nchunk - 1) * _G_K, _G_K)],
                wsems[bl])
            if tail:
                cpt.wait()
                wct = pltpu.async_copy(rowst, out_hbm.at[pl.ds(toff, tail)],
                                       wsemt)
            for b in (0, 1):
                if wcps[b] is not None:
                    wcps[b].wait()
            if wct is not None:
                wct.wait()

        one(snd_hbm, out_s_hbm)
        one(rcv_hbm, out_r_hbm)

    return pl.kernel(
        body,
        out_type=(jax.ShapeDtypeStruct((slab_edges, width), jnp.float32),
                  jax.ShapeDtypeStruct((slab_edges, width), jnp.float32)),
        mesh=_sc_mesh(),
        scratch_types=[
            pltpu.VMEM((_G_K,), jnp.int32),
            pltpu.VMEM((_G_K,), jnp.int32),
            pltpu.VMEM((_G_K, width), jnp.float32),
            pltpu.VMEM((_G_K, width), jnp.float32),
            pltpu.VMEM((max(tail, 8),), jnp.int32),
            pltpu.VMEM((max(tail, 8), width), jnp.float32),
            pltpu.SemaphoreType.DMA,
            pltpu.SemaphoreType.DMA,
            pltpu.SemaphoreType.DMA,
            pltpu.SemaphoreType.DMA,
            pltpu.SemaphoreType.DMA,
            pltpu.SemaphoreType.DMA,
        ],
    )


_S_H = D // NC
_S_K = 128
_S_RPT = 624
_S_REM = N_NODES - NS * _S_RPT
_Z_R = 104


@functools.lru_cache(maxsize=None)
def _sc_scatter_kernel(bases, slab_edges):
    ng = len(bases)
    tpg = NS // ng
    ept = slab_edges // tpg
    nchunk = ept // _S_K
    tail = ept - nchunk * _S_K

    def body(*refs):
        mij_hbms = refs[:ng]
        rcv_hbm, zeros_hbm, out_hbm = refs[ng:ng + 3]
        idx_v, rows_v, idxt_v, rowst_v, zbuf, acc_sh, sem = refs[ng + 3:]
        c = lax.axis_index("c")
        s = lax.axis_index("s")
        col = c * _S_H
        pltpu.sync_copy(zeros_hbm, zbuf)
        for r in range(_S_RPT // _Z_R):
            pltpu.sync_copy(zbuf, acc_sh.at[pl.ds(s * _S_RPT + r * _Z_R,
                                                  _Z_R)])
        @pl.when(s == 0)
        def _():
            pltpu.sync_copy(zbuf.at[pl.ds(0, _S_REM)],
                            acc_sh.at[pl.ds(NS * _S_RPT, _S_REM)])
        plsc.subcore_barrier()

        def scan(mij_hbm, ebase, loc):
            def chunk(j, _):
                off = loc + j * _S_K
                pltpu.sync_copy(rcv_hbm.at[pl.ds(ebase + off, _S_K)], idx_v)
                pltpu.sync_copy(mij_hbm.at[pl.ds(off, _S_K), pl.ds(col, _S_H)],
                                rows_v)
                pltpu.sync_copy(rows_v, acc_sh.at[idx_v], add=True)
                return _
            lax.fori_loop(0, nchunk, chunk, 0)
            if tail:
                toff = loc + nchunk * _S_K
                pltpu.sync_copy(rcv_hbm.at[pl.ds(ebase + toff, tail)], idxt_v)
                pltpu.sync_copy(mij_hbm.at[pl.ds(toff, tail), pl.ds(col, _S_H)],
                                rowst_v)
                pltpu.sync_copy(rowst_v, acc_sh.at[idxt_v], add=True)

        for gi in range(ng):
            @pl.when((s >= gi * tpg) & (s < (gi + 1) * tpg))
            def _(gi=gi):
                scan(mij_hbms[gi], bases[gi], (s - gi * tpg) * ept)

        plsc.subcore_barrier()
        pltpu.sync_copy(acc_sh.at[pl.ds(s * _S_RPT, _S_RPT)],
                        out_hbm.at[pl.ds(s * _S_RPT, _S_RPT), pl.ds(col, _S_H)])
        @pl.when(s == 0)
        def _():
            pltpu.sync_copy(
                acc_sh.at[pl.ds(NS * _S_RPT, _S_REM)],
                out_hbm.at[pl.ds(NS * _S_RPT, _S_REM), pl.ds(col, _S_H)])

    return pl.kernel(
        body,
        out_type=jax.ShapeDtypeStruct((N_NODES, D), jnp.float32),
        mesh=_sc_mesh(),
        scratch_types=[
            pltpu.VMEM((_S_K,), jnp.int32),
            pltpu.VMEM((_S_K, _S_H), jnp.float32),
            pltpu.VMEM((max(tail, 8),), jnp.int32),
            pltpu.VMEM((max(tail, 8), _S_H), jnp.float32),
            pltpu.VMEM((_Z_R, _S_H), jnp.float32),
            pltpu.VMEM_SHARED((N_NODES, _S_H), jnp.float32),
            pltpu.SemaphoreType.DMA,
        ],
    )


_P_BLK = 2000
_HW = D // 2


def _pack_body(nf_r, o_r):
    x = nf_r[...]
    lo = x[:, :_HW].astype(jnp.bfloat16).astype(jnp.float32)
    hi = x[:, _HW:].astype(jnp.bfloat16).astype(jnp.float32)
    lo_u = lax.bitcast_convert_type(lo, jnp.uint32)
    hi_u = lax.bitcast_convert_type(hi, jnp.uint32)
    w = hi_u | (lo_u >> 16)
    o_r[...] = lax.bitcast_convert_type(w, jnp.float32)


def _pack(nf):
    grid = (N_NODES // _P_BLK,)
    return pl.pallas_call(
        _pack_body,
        grid=grid,
        in_specs=[pl.BlockSpec((_P_BLK, D), lambda i: (i, 0))],
        out_specs=pl.BlockSpec((_P_BLK, _HW), lambda i: (i, 0)),
        out_shape=jax.ShapeDtypeStruct((N_NODES, _HW), jnp.float32),
    )(nf)


def _unpack(w_f32):
    u = lax.bitcast_convert_type(w_f32, jnp.uint32)
    hi = lax.bitcast_convert_type(u & jnp.uint32(0xFFFF0000), jnp.float32)
    lo = lax.bitcast_convert_type(u << 16, jnp.float32)
    return jnp.concatenate([lo, hi], axis=1).astype(jnp.bfloat16)


_E_BLK = 1000


def _edge_mlp_body(sf_r, rf_r, ef_r, w1s_r, w1r_r, w1e_r, b1_r, w2_r, b2_r,
                   o_r):
    sfx = _unpack(sf_r[...])
    rfx = _unpack(rf_r[...])
    ef_bf = ef_r[...].astype(jnp.bfloat16)
    h = (jnp.dot(sfx, w1s_r[...], preferred_element_type=jnp.float32)
         + jnp.dot(rfx, w1r_r[...], preferred_element_type=jnp.float32)
         + jnp.dot(ef_bf, w1e_r[...], preferred_element_type=jnp.float32)
         + b1_r[...])
    h = jax.nn.gelu(h).astype(jnp.bfloat16)
    o_r[...] = (jnp.dot(h, w2_r[...], preferred_element_type=jnp.float32)
                + b2_r[...]) * 0.25


def _edge_mlp(sf, rf, ef, w1s, w1r, w1e, b1, w2, b2, slab_base, slab_edges):
    nin = 3 * D
    grid = (slab_edges // _E_BLK,)
    base_blk = slab_base // _E_BLK
    blk = lambda i: (i, 0)
    efblk = lambda i: (base_blk + i, 0)
    fixed = lambda i: (0, 0)
    return pl.pallas_call(
        _edge_mlp_body,
        grid=grid,
        in_specs=[
            pl.BlockSpec((_E_BLK, _HW), blk),
            pl.BlockSpec((_E_BLK, _HW), blk),
            pl.BlockSpec((_E_BLK, D), efblk),
            pl.BlockSpec((D, nin), fixed),
            pl.BlockSpec((D, nin), fixed),
            pl.BlockSpec((D, nin), fixed),
            pl.BlockSpec((1, nin), fixed),
            pl.BlockSpec((nin, D), fixed),
            pl.BlockSpec((1, D), fixed),
        ],
        out_specs=pl.BlockSpec((_E_BLK, D), blk),
        out_shape=jax.ShapeDtypeStruct((slab_edges, D), jnp.float32),
    )(sf, rf, ef, w1s, w1r, w1e, b1, w2, b2)


_N_BLK = 2000


def _node_mlp_body(*refs):
    o_r = refs[-1]
    parts = refs[:-7]
    nf_r, w1m_r, w1x_r, b1_r, w2_r, b2_r = refs[-7:-1]
    mi = parts[0][...]
    for p in parts[1:]:
        mi = mi + p[...]
    g = (jnp.dot(mi, w1m_r[...], preferred_element_type=jnp.float32)
         + jnp.dot(nf_r[...], w1x_r[...], preferred_element_type=jnp.float32)
         + b1_r[...])
    g = jax.nn.gelu(g)
    o_r[...] = (jnp.dot(g, w2_r[...], preferred_element_type=jnp.float32)
                + b2_r[...])


def _node_mlp(mi_parts, nf, w1m, w1x, b1, w2, b2):
    nin = 2 * D
    grid = (N_NODES // _N_BLK,)
    blk = lambda i: (i, 0)
    fixed = lambda i: (0, 0)
    return pl.pallas_call(
        _node_mlp_body,
        grid=grid,
        in_specs=[pl.BlockSpec((_N_BLK, D), blk)] * len(mi_parts) + [
            pl.BlockSpec((_N_BLK, D), blk),
            pl.BlockSpec((D, nin), fixed),
            pl.BlockSpec((D, nin), fixed),
            pl.BlockSpec((1, nin), fixed),
            pl.BlockSpec((nin, D), fixed),
            pl.BlockSpec((1, D), fixed),
        ],
        out_specs=pl.BlockSpec((_N_BLK, D), blk),
        out_shape=jax.ShapeDtypeStruct((N_NODES, D), jnp.float32),
    )(*mi_parts, nf, w1m, w1x, b1, w2, b2)


def kernel(node_features, edge_features, senders, receivers,
           W1e, b1e, W2e, b2e, W1n, b1n, W2n, b2n):
    nf = node_features[0]
    ef = edge_features[0]
    snd = senders[0].astype(jnp.int32)
    rcv = receivers[0].astype(jnp.int32)

    nf_words = _pack(nf)
    zeros = jnp.zeros((_Z_R, _S_H), jnp.float32)
    bf = jnp.bfloat16
    w1s, w1r, w1x = (W1e[:D].astype(bf), W1e[D:2 * D].astype(bf),
                     W1e[2 * D:].astype(bf))
    b1 = b1e.reshape(1, -1)
    w2 = W2e.astype(bf)
    b2 = b2e.reshape(1, -1)

    slab = N_EDGES // _NSLAB
    m_ijs = []
    for i in range(_NSLAB):
        base = i * slab
        sf_w, rf_w = _sc_gather_kernel(_HW, base, slab)(nf_words, snd, rcv)
        m_ijs.append(_edge_mlp(sf_w, rf_w, ef, w1s, w1r, w1x, b1, w2, b2,
                               base, slab))

    parts = []
    for grp in _SGROUPS:
        bases = tuple(i * slab for i in grp)
        parts.append(_sc_scatter_kernel(bases, slab)(
            *[m_ijs[i] for i in grp], rcv, zeros))

    out = _node_mlp(parts, nf,
                    W1n[:D], W1n[D:],
                    b1n.reshape(1, -1), W2n, b2n.reshape(1, -1))
    return out[None]


# speedup vs baseline: 3.6708x; 1.0378x over previous
import functools

import jax
import jax.numpy as jnp
from jax import lax
from jax.experimental import pallas as pl
from jax.experimental.pallas import tpu as pltpu
from jax.experimental.pallas import tpu_sc as plsc

N_NODES = 10000
N_EDGES = 160000
D = 256

NC = 2
NS = 16
NW = NC * NS
_NSLAB = 5
_SGROUPS = ((0, 1), (2, 3), (4,))

@functools.lru_cache(maxsize=None)
def _sc_mesh():
    return plsc.VectorSubcoreMesh(core_axis_name="c", subcore_axis_name="s",
                                  num_cores=NC, num_subcores=NS)


_G_K = 128


@functools.lru_cache(maxsize=None)
def _sc_gather_kernel(width, slab_base, slab_edges):
    epw = slab_edges // NW
    nchunk = epw // _G_K
    tail = epw - nchunk * _G_K

    def body(nf_hbm, snd_hbm, rcv_hbm, out_s_hbm, out_r_hbm,
             idx0, idx1, rows0, rows1, idxt, rowst,
             sem0, sem1, semt, wsem0, wsem1, wsemt):
        idxs = (idx0, idx1)
        rows = (rows0, rows1)
        sems = (sem0, sem1)
        wsems = (wsem0, wsem1)
        wid = lax.axis_index("s") * NC + lax.axis_index("c")
        loc = wid * epw

        def one(idx_hbm, out_hbm):
            cps = [None, None]
            wcps = [None, None]
            for j in range(nchunk):
                b = j & 1
                if wcps[b] is not None:
                    wcps[b].wait()
                    wcps[b] = None
                pltpu.sync_copy(
                    idx_hbm.at[pl.ds(slab_base + loc + j * _G_K, _G_K)],
                    idxs[b])
                cps[b] = pltpu.async_copy(nf_hbm.at[idxs[b]], rows[b],
                                          sems[b])
                if j:
                    cps[1 - b].wait()
                    wcps[1 - b] = pltpu.async_copy(
                        rows[1 - b],
                        out_hbm.at[pl.ds(loc + (j - 1) * _G_K, _G_K)],
                        wsems[1 - b])
            cpt = wct = None
            if tail:
                toff = loc + nchunk * _G_K
                pltpu.sync_copy(idx_hbm.at[pl.ds(slab_base + toff, tail)],
                                idxt)
                cpt = pltpu.async_copy(nf_hbm.at[idxt], rowst, semt)
            bl = (nchunk - 1) & 1
            cps[bl].wait()
            wcps[bl] = pltpu.async_copy(
                rows[bl], out_hbm.at[pl.ds(loc + (nchunk - 1) * _G_K, _G_K)],
                wsems[bl])
            if tail:
                cpt.wait()
                wct = pltpu.async_copy(rowst, out_hbm.at[pl.ds(toff, tail)],
                                       wsemt)
            for b in (0, 1):
                if wcps[b] is not None:
                    wcps[b].wait()
            if wct is not None:
                wct.wait()

        one(snd_hbm, out_s_hbm)
        one(rcv_hbm, out_r_hbm)

    return pl.kernel(
        body,
        out_type=(jax.ShapeDtypeStruct((slab_edges, width), jnp.float32),
                  jax.ShapeDtypeStruct((slab_edges, width), jnp.float32)),
        mesh=_sc_mesh(),
        scratch_types=[
            pltpu.VMEM((_G_K,), jnp.int32),
            pltpu.VMEM((_G_K,), jnp.int32),
            pltpu.VMEM((_G_K, width), jnp.float32),
            pltpu.VMEM((_G_K, width), jnp.float32),
            pltpu.VMEM((max(tail, 8),), jnp.int32),
            pltpu.VMEM((max(tail, 8), width), jnp.float32),
            pltpu.SemaphoreType.DMA,
            pltpu.SemaphoreType.DMA,
            pltpu.SemaphoreType.DMA,
            pltpu.SemaphoreType.DMA,
            pltpu.SemaphoreType.DMA,
            pltpu.SemaphoreType.DMA,
        ],
    )


_S_H = D // NC
_S_K = 128
_S_RPT = 624
_S_REM = N_NODES - NS * _S_RPT
_Z_R = 8


@functools.lru_cache(maxsize=None)
def _sc_scatter_kernel(bases, slab_edges):
    ng = len(bases)
    tpg = NS // ng
    ept = slab_edges // tpg
    nchunk = ept // _S_K
    tail = ept - nchunk * _S_K

    def body(*refs):
        mij_hbms = refs[:ng]
        rcv_hbm, zeros_hbm, out_hbm = refs[ng:ng + 3]
        (idx0, idx1, rows0, rows1, idxt_v, rowst_v, zbuf, acc_sh,
         sem0, sem1, semt) = refs[ng + 3:]
        idxs = (idx0, idx1)
        rows = (rows0, rows1)
        sems = (sem0, sem1)
        c = lax.axis_index("c")
        s = lax.axis_index("s")
        col = c * _S_H
        pltpu.sync_copy(zeros_hbm, zbuf)
        for r in range(_S_RPT // _Z_R):
            pltpu.sync_copy(zbuf, acc_sh.at[pl.ds(s * _S_RPT + r * _Z_R,
                                                  _Z_R)])
        @pl.when(s == 0)
        def _():
            for r in range(_S_REM // _Z_R):
                pltpu.sync_copy(
                    zbuf, acc_sh.at[pl.ds(NS * _S_RPT + r * _Z_R, _Z_R)])
        plsc.subcore_barrier()

        def scan(mij_hbm, ebase, loc):
            cps = [None, None]
            for j in range(nchunk):
                b = j & 1
                pltpu.sync_copy(rcv_hbm.at[pl.ds(ebase + loc + j * _S_K,
                                                 _S_K)], idxs[b])
                cps[b] = pltpu.async_copy(
                    mij_hbm.at[pl.ds(loc + j * _S_K, _S_K),
                               pl.ds(col, _S_H)],
                    rows[b], sems[b])
                if j:
                    cps[1 - b].wait()
                    pltpu.sync_copy(rows[1 - b], acc_sh.at[idxs[1 - b]],
                                    add=True)
            cpt = None
            if tail:
                toff = loc + nchunk * _S_K
                pltpu.sync_copy(rcv_hbm.at[pl.ds(ebase + toff, tail)], idxt_v)
                cpt = pltpu.async_copy(
                    mij_hbm.at[pl.ds(toff, tail), pl.ds(col, _S_H)],
                    rowst_v, semt)
            bl = (nchunk - 1) & 1
            cps[bl].wait()
            pltpu.sync_copy(rows[bl], acc_sh.at[idxs[bl]], add=True)
            if tail:
                cpt.wait()
                pltpu.sync_copy(rowst_v, acc_sh.at[idxt_v], add=True)

        for gi in range(ng):
            @pl.when((s >= gi * tpg) & (s < (gi + 1) * tpg))
            def _(gi=gi):
                scan(mij_hbms[gi], bases[gi], (s - gi * tpg) * ept)

        plsc.subcore_barrier()
        pltpu.sync_copy(acc_sh.at[pl.ds(s * _S_RPT, _S_RPT)],
                        out_hbm.at[pl.ds(s * _S_RPT, _S_RPT), pl.ds(col, _S_H)])
        @pl.when(s == 0)
        def _():
            pltpu.sync_copy(
                acc_sh.at[pl.ds(NS * _S_RPT, _S_REM)],
                out_hbm.at[pl.ds(NS * _S_RPT, _S_REM), pl.ds(col, _S_H)])

    return pl.kernel(
        body,
        out_type=jax.ShapeDtypeStruct((N_NODES, D), jnp.float32),
        mesh=_sc_mesh(),
        scratch_types=[
            pltpu.VMEM((_S_K,), jnp.int32),
            pltpu.VMEM((_S_K,), jnp.int32),
            pltpu.VMEM((_S_K, _S_H), jnp.float32),
            pltpu.VMEM((_S_K, _S_H), jnp.float32),
            pltpu.VMEM((max(tail, 8),), jnp.int32),
            pltpu.VMEM((max(tail, 8), _S_H), jnp.float32),
            pltpu.VMEM((_Z_R, _S_H), jnp.float32),
            pltpu.VMEM_SHARED((N_NODES, _S_H), jnp.float32),
            pltpu.SemaphoreType.DMA,
            pltpu.SemaphoreType.DMA,
            pltpu.SemaphoreType.DMA,
        ],
    )


_P_BLK = 2000
_HW = D // 2


def _pack_body(nf_r, o_r):
    x = nf_r[...]
    lo = x[:, :_HW].astype(jnp.bfloat16).astype(jnp.float32)
    hi = x[:, _HW:].astype(jnp.bfloat16).astype(jnp.float32)
    lo_u = lax.bitcast_convert_type(lo, jnp.uint32)
    hi_u = lax.bitcast_convert_type(hi, jnp.uint32)
    w = hi_u | (lo_u >> 16)
    o_r[...] = lax.bitcast_convert_type(w, jnp.float32)


def _pack(nf):
    grid = (N_NODES // _P_BLK,)
    return pl.pallas_call(
        _pack_body,
        grid=grid,
        in_specs=[pl.BlockSpec((_P_BLK, D), lambda i: (i, 0))],
        out_specs=pl.BlockSpec((_P_BLK, _HW), lambda i: (i, 0)),
        out_shape=jax.ShapeDtypeStruct((N_NODES, _HW), jnp.float32),
    )(nf)


def _unpack(w_f32):
    u = lax.bitcast_convert_type(w_f32, jnp.uint32)
    hi = lax.bitcast_convert_type(u & jnp.uint32(0xFFFF0000), jnp.float32)
    lo = lax.bitcast_convert_type(u << 16, jnp.float32)
    return jnp.concatenate([lo, hi], axis=1).astype(jnp.bfloat16)


_E_BLK = 1000


def _edge_mlp_body(sf_r, rf_r, ef_r, w1s_r, w1r_r, w1e_r, b1_r, w2_r, b2_r,
                   o_r):
    sfx = _unpack(sf_r[...])
    rfx = _unpack(rf_r[...])
    ef_bf = ef_r[...].astype(jnp.bfloat16)
    h = (jnp.dot(sfx, w1s_r[...], preferred_element_type=jnp.float32)
         + jnp.dot(rfx, w1r_r[...], preferred_element_type=jnp.float32)
         + jnp.dot(ef_bf, w1e_r[...], preferred_element_type=jnp.float32)
         + b1_r[...])
    h = jax.nn.gelu(h).astype(jnp.bfloat16)
    o_r[...] = (jnp.dot(h, w2_r[...], preferred_element_type=jnp.float32)
                + b2_r[...]) * 0.25


def _edge_mlp(sf, rf, ef, w1s, w1r, w1e, b1, w2, b2, slab_base, slab_edges):
    nin = 3 * D
    grid = (slab_edges // _E_BLK,)
    base_blk = slab_base // _E_BLK
    blk = lambda i: (i, 0)
    efblk = lambda i: (base_blk + i, 0)
    fixed = lambda i: (0, 0)
    return pl.pallas_call(
        _edge_mlp_body,
        grid=grid,
        in_specs=[
            pl.BlockSpec((_E_BLK, _HW), blk),
            pl.BlockSpec((_E_BLK, _HW), blk),
            pl.BlockSpec((_E_BLK, D), efblk),
            pl.BlockSpec((D, nin), fixed),
            pl.BlockSpec((D, nin), fixed),
            pl.BlockSpec((D, nin), fixed),
            pl.BlockSpec((1, nin), fixed),
            pl.BlockSpec((nin, D), fixed),
            pl.BlockSpec((1, D), fixed),
        ],
        out_specs=pl.BlockSpec((_E_BLK, D), blk),
        out_shape=jax.ShapeDtypeStruct((slab_edges, D), jnp.float32),
    )(sf, rf, ef, w1s, w1r, w1e, b1, w2, b2)


_N_BLK = 2000


def _node_mlp_body(*refs):
    o_r = refs[-1]
    parts = refs[:-7]
    nf_r, w1m_r, w1x_r, b1_r, w2_r, b2_r = refs[-7:-1]
    mi = parts[0][...]
    for p in parts[1:]:
        mi = mi + p[...]
    g = (jnp.dot(mi, w1m_r[...], preferred_element_type=jnp.float32)
         + jnp.dot(nf_r[...], w1x_r[...], preferred_element_type=jnp.float32)
         + b1_r[...])
    g = jax.nn.gelu(g)
    o_r[...] = (jnp.dot(g, w2_r[...], preferred_element_type=jnp.float32)
                + b2_r[...])


def _node_mlp(mi_parts, nf, w1m, w1x, b1, w2, b2):
    nin = 2 * D
    grid = (N_NODES // _N_BLK,)
    blk = lambda i: (i, 0)
    fixed = lambda i: (0, 0)
    return pl.pallas_call(
        _node_mlp_body,
        grid=grid,
        in_specs=[pl.BlockSpec((_N_BLK, D), blk)] * len(mi_parts) + [
            pl.BlockSpec((_N_BLK, D), blk),
            pl.BlockSpec((D, nin), fixed),
            pl.BlockSpec((D, nin), fixed),
            pl.BlockSpec((1, nin), fixed),
            pl.BlockSpec((nin, D), fixed),
            pl.BlockSpec((1, D), fixed),
        ],
        out_specs=pl.BlockSpec((_N_BLK, D), blk),
        out_shape=jax.ShapeDtypeStruct((N_NODES, D), jnp.float32),
    )(*mi_parts, nf, w1m, w1x, b1, w2, b2)


def kernel(node_features, edge_features, senders, receivers,
           W1e, b1e, W2e, b2e, W1n, b1n, W2n, b2n):
    nf = node_features[0]
    ef = edge_features[0]
    snd = senders[0].astype(jnp.int32)
    rcv = receivers[0].astype(jnp.int32)

    nf_words = _pack(nf)
    zeros = jnp.zeros((_Z_R, _S_H), jnp.float32)
    bf = jnp.bfloat16
    w1s, w1r, w1x = (W1e[:D].astype(bf), W1e[D:2 * D].astype(bf),
                     W1e[2 * D:].astype(bf))
    b1 = b1e.reshape(1, -1)
    w2 = W2e.astype(bf)
    b2 = b2e.reshape(1, -1)

    slab = N_EDGES // _NSLAB
    m_ijs = []
    for i in range(_NSLAB):
        base = i * slab
        sf_w, rf_w = _sc_gather_kernel(_HW, base, slab)(nf_words, snd, rcv)
        m_ijs.append(_edge_mlp(sf_w, rf_w, ef, w1s, w1r, w1x, b1, w2, b2,
                               base, slab))

    parts = []
    for grp in _SGROUPS:
        bases = tuple(i * slab for i in grp)
        parts.append(_sc_scatter_kernel(bases, slab)(
            *[m_ijs[i] for i in grp], rcv, zeros))

    out = _node_mlp(parts, nf,
                    W1n[:D], W1n[D:],
                    b1n.reshape(1, -1), W2n, b2n.reshape(1, -1))
    return out[None]
